# baseline (speedup 1.0000x reference)
import jax
import jax.numpy as jnp
from jax import lax
from jax.experimental import pallas as pl
from jax.experimental.pallas import tpu as pltpu
from jax.experimental.pallas import tpu_sc as plsc

_NC = 2
_NS = 16
_NW = _NC * _NS
_GW = 256
_DEPTH = 4


def kernel(x, E_absolute_position):
    B, H = x.shape
    N, D = E_absolute_position.shape
    num_indices = B * H
    per_w = num_indices // _NW
    n_chunks = per_w // _GW
    n_groups = n_chunks // _DEPTH
    assert per_w * _NW == num_indices
    assert n_groups * _DEPTH * _GW == per_w

    idx = x.reshape(_NW, n_chunks, _GW).astype(jnp.int32)

    mesh = plsc.VectorSubcoreMesh(core_axis_name="c", subcore_axis_name="s")

    scratch = (
        [pltpu.VMEM((n_chunks, _GW), jnp.int32)]
        + [pltpu.VMEM((_GW, D), jnp.float32) for _ in range(_DEPTH)]
        + [pltpu.SemaphoreType.DMA for _ in range(2 * _DEPTH)]
    )

    @pl.kernel(
        out_type=jax.ShapeDtypeStruct((num_indices, D),
                                      E_absolute_position.dtype),
        mesh=mesh,
        compiler_params=pltpu.CompilerParams(use_tc_tiling_on_sc=False),
        scratch_types=scratch,
    )
    def gather_kernel(table_hbm, idx_hbm, out_hbm, idx_v, *scr):
        rows = scr[:_DEPTH]
        gsem = scr[_DEPTH:2 * _DEPTH]
        wsem = scr[2 * _DEPTH:]
        wid = lax.axis_index("s") * _NC + lax.axis_index("c")
        base = wid * per_w

        pltpu.sync_copy(idx_hbm.at[wid], idx_v)

        def start_gather(c, k):
            pltpu.make_async_copy(table_hbm.at[idx_v.at[c]], rows[k],
                                  gsem[k]).start()

        def wait_gather(c, k):
            pltpu.make_async_copy(table_hbm.at[idx_v.at[c]], rows[k],
                                  gsem[k]).wait()

        def out_copy(c, k):
            return pltpu.make_async_copy(
                rows[k], out_hbm.at[pl.ds(base + c * _GW, _GW)], wsem[k])

        for k in range(_DEPTH):
            start_gather(k, k)

        @pl.loop(0, n_groups)
        def _(t):
            c0 = t * _DEPTH
            for k in range(_DEPTH):
                wait_gather(c0 + k, k)
                out_copy(c0 + k, k).start()
            for k in range(_DEPTH):
                cn = lax.rem(c0 + k + _DEPTH, n_chunks)
                out_copy(c0 + k, k).wait()
                start_gather(cn, k)

        for k in range(_DEPTH):
            wait_gather(k, k)

    out = gather_kernel(E_absolute_position, idx)
    return out.reshape(B, H, D)


# speedup vs baseline: 1.2214x; 1.2214x over previous
import jax
import jax.numpy as jnp
from jax import lax
from jax.experimental import pallas as pl
from jax.experimental.pallas import tpu as pltpu
from jax.experimental.pallas import tpu_sc as plsc

_NC = 2
_NS = 16
_NW = _NC * _NS
_GW = 128
_DEPTH = 4


def kernel(x, E_absolute_position):
    B, H = x.shape
    N, D = E_absolute_position.shape
    num_indices = B * H
    per_w = num_indices // _NW
    n_chunks = per_w // _GW
    n_groups = n_chunks // _DEPTH
    assert per_w * _NW == num_indices
    assert n_groups * _DEPTH * _GW == per_w

    idx = x.reshape(_NW, n_chunks, _GW).astype(jnp.int32)
    table = jnp.pad(E_absolute_position, ((0, 0), (0, 128 - D)))

    mesh = plsc.VectorSubcoreMesh(core_axis_name="c", subcore_axis_name="s")

    scratch = (
        [pltpu.VMEM((n_chunks, _GW), jnp.int32)]
        + [pltpu.VMEM((_GW, 128), jnp.float32) for _ in range(_DEPTH)]
        + [pltpu.SemaphoreType.DMA for _ in range(2 * _DEPTH)]
    )

    @pl.kernel(
        out_type=jax.ShapeDtypeStruct((num_indices, 128),
                                      E_absolute_position.dtype),
        mesh=mesh,
        scratch_types=scratch,
    )
    def gather_kernel(table_hbm, idx_hbm, out_hbm, idx_v, *scr):
        rows = scr[:_DEPTH]
        gsem = scr[_DEPTH:2 * _DEPTH]
        wsem = scr[2 * _DEPTH:]
        wid = lax.axis_index("s") * _NC + lax.axis_index("c")
        base = wid * per_w

        pltpu.sync_copy(idx_hbm.at[wid], idx_v)

        def start_gather(c, k):
            pltpu.make_async_copy(table_hbm.at[idx_v.at[c]], rows[k],
                                  gsem[k]).start()

        def wait_gather(c, k):
            pltpu.make_async_copy(table_hbm.at[idx_v.at[c]], rows[k],
                                  gsem[k]).wait()

        def out_copy(c, k):
            return pltpu.make_async_copy(
                rows[k], out_hbm.at[pl.ds(base + c * _GW, _GW)], wsem[k])

        for k in range(_DEPTH):
            start_gather(k, k)

        @pl.loop(0, n_groups)
        def _(t):
            c0 = t * _DEPTH
            for k in range(_DEPTH):
                wait_gather(c0 + k, k)
                out_copy(c0 + k, k).start()
            for k in range(_DEPTH):
                cn = lax.rem(c0 + k + _DEPTH, n_chunks)
                out_copy(c0 + k, k).wait()
                start_gather(cn, k)

        for k in range(_DEPTH):
            wait_gather(k, k)

    out = gather_kernel(table, idx)
    return out[:, :D].reshape(B, H, D)
